# baseline (speedup 1.0000x reference)
import jax
import jax.numpy as jnp
from jax.experimental import pallas as pl
from jax.experimental.pallas import tpu as pltpu

_N, _D = 100000, 128
_NCHUNK = 8
_CH = _N // _NCHUNK


def _dma_body(x_ref, o_ref, sems):
    for i in range(_NCHUNK):
        pltpu.make_async_copy(
            x_ref.at[pl.ds(i * _CH, _CH)],
            o_ref.at[pl.ds(i * _CH, _CH)],
            sems.at[i],
        ).start()
    for i in range(_NCHUNK):
        pltpu.make_async_copy(
            x_ref.at[pl.ds(i * _CH, _CH)],
            o_ref.at[pl.ds(i * _CH, _CH)],
            sems.at[i],
        ).wait()


def kernel(data_list):
    return pl.pallas_call(
        _dma_body,
        in_specs=[pl.BlockSpec(memory_space=pltpu.MemorySpace.HBM)],
        out_specs=pl.BlockSpec(memory_space=pltpu.MemorySpace.HBM),
        out_shape=jax.ShapeDtypeStruct((_N, _D), jnp.float32),
        scratch_shapes=[pltpu.SemaphoreType.DMA((_NCHUNK,))],
    )(data_list)


# speedup vs baseline: 42.6232x; 42.6232x over previous
import jax
import jax.numpy as jnp
from jax.experimental import pallas as pl

_N, _D = 100000, 128
_BLOCK = 4000


def _copy_body(x_ref, o_ref):
    o_ref[...] = x_ref[...]


def kernel(data_list):
    return pl.pallas_call(
        _copy_body,
        grid=(_N // _BLOCK,),
        in_specs=[pl.BlockSpec((_BLOCK, _D), lambda i: (i, 0))],
        out_specs=pl.BlockSpec((_BLOCK, _D), lambda i: (i, 0)),
        out_shape=jax.ShapeDtypeStruct((_N, _D), jnp.float32),
    )(data_list)


# speedup vs baseline: 47.1869x; 1.1071x over previous
import jax
import jax.numpy as jnp
from jax.experimental import pallas as pl

_N, _D = 100000, 128
_BLOCK = 10000


def _copy_body(x_ref, o_ref):
    o_ref[...] = x_ref[...]


def kernel(data_list):
    return pl.pallas_call(
        _copy_body,
        grid=(_N // _BLOCK,),
        in_specs=[pl.BlockSpec((_BLOCK, _D), lambda i: (i, 0))],
        out_specs=pl.BlockSpec((_BLOCK, _D), lambda i: (i, 0)),
        out_shape=jax.ShapeDtypeStruct((_N, _D), jnp.float32),
    )(data_list)


# speedup vs baseline: 49.4237x; 1.0474x over previous
import jax
import jax.numpy as jnp
from jax.experimental import pallas as pl

_N, _D = 100000, 128
_BLOCK = 20000


def _copy_body(x_ref, o_ref):
    o_ref[...] = x_ref[...]


def kernel(data_list):
    return pl.pallas_call(
        _copy_body,
        grid=(_N // _BLOCK,),
        in_specs=[pl.BlockSpec((_BLOCK, _D), lambda i: (i, 0))],
        out_specs=pl.BlockSpec((_BLOCK, _D), lambda i: (i, 0)),
        out_shape=jax.ShapeDtypeStruct((_N, _D), jnp.float32),
    )(data_list)
